# baseline (speedup 1.0000x reference)
import jax
import jax.numpy as jnp
from jax.experimental import pallas as pl
from jax.experimental.pallas import tpu as pltpu

_B = 128
_D = 64
_BLOCK = 8192


def _flash_kernel(x_ref, k_ref, v_ref, o_ref, m_ref, l_ref, acc_ref):
    i = pl.program_id(0)
    n = pl.num_programs(0)

    @pl.when(i == 0)
    def _init():
        m_ref[...] = jnp.full_like(m_ref, -jnp.inf)
        l_ref[...] = jnp.zeros_like(l_ref)
        acc_ref[...] = jnp.zeros_like(acc_ref)

    x = x_ref[...]
    kb = k_ref[...]
    s = jax.lax.dot_general(
        x, kb, (((1,), (0,)), ((), ())),
        preferred_element_type=jnp.float32)

    m_prev = m_ref[...]
    m_cur = jnp.max(s, axis=1, keepdims=True)
    m_new = jnp.maximum(m_prev, m_cur)

    alpha = jnp.exp(m_prev - m_new)
    p = jnp.exp(s - m_new[:, 0:1])

    l_cur = jnp.sum(p, axis=1, keepdims=True)
    l_ref[...] = l_ref[...] * alpha + l_cur
    m_ref[...] = m_new

    pv = jax.lax.dot_general(
        p, v_ref[...], (((1,), (1,)), ((), ())),
        preferred_element_type=jnp.float32)
    acc_ref[...] = acc_ref[...] * alpha[:, 0:1] + pv

    @pl.when(i == n - 1)
    def _finish():
        o_ref[...] = acc_ref[...] / l_ref[...][:, 0:1]


def kernel(x, memory_keys, memory_values):
    kT = memory_keys.T
    vT = memory_values.T
    m_total = memory_keys.shape[0]
    grid = (m_total // _BLOCK,)
    return pl.pallas_call(
        _flash_kernel,
        grid=grid,
        in_specs=[
            pl.BlockSpec((_B, _D), lambda i: (0, 0)),
            pl.BlockSpec((_D, _BLOCK), lambda i: (0, i)),
            pl.BlockSpec((_D, _BLOCK), lambda i: (0, i)),
        ],
        out_specs=pl.BlockSpec((_B, _D), lambda i: (0, 0)),
        out_shape=jax.ShapeDtypeStruct((_B, _D), jnp.float32),
        scratch_shapes=[
            pltpu.VMEM((_B, 128), jnp.float32),
            pltpu.VMEM((_B, 128), jnp.float32),
            pltpu.VMEM((_B, _D), jnp.float32),
        ],
        compiler_params=pltpu.CompilerParams(
            dimension_semantics=("parallel",),
        ),
    )(x, kT, vT)


# speedup vs baseline: 1.1593x; 1.1593x over previous
import jax
import jax.numpy as jnp
from jax.experimental import pallas as pl
from jax.experimental.pallas import tpu as pltpu

_B = 128
_D = 64
_BLOCK = 32768


def _flash_kernel(x_ref, k_ref, v_ref, o_ref, m_ref, l_ref, acc_ref):
    i = pl.program_id(0)
    n = pl.num_programs(0)

    @pl.when(i == 0)
    def _init():
        m_ref[...] = jnp.full_like(m_ref, -jnp.inf)
        l_ref[...] = jnp.zeros_like(l_ref)
        acc_ref[...] = jnp.zeros_like(acc_ref)

    x = x_ref[...]
    kb = k_ref[...]
    s = jax.lax.dot_general(
        x, kb, (((1,), (0,)), ((), ())),
        preferred_element_type=jnp.float32)

    m_prev = m_ref[...]
    m_cur = jnp.max(s, axis=1, keepdims=True)
    m_new = jnp.maximum(m_prev, m_cur)

    alpha = jnp.exp(m_prev - m_new)
    p = jnp.exp(s - m_new[:, 0:1])

    l_cur = jnp.sum(p, axis=1, keepdims=True)
    l_ref[...] = l_ref[...] * alpha + l_cur
    m_ref[...] = m_new

    pv = jax.lax.dot_general(
        p, v_ref[...], (((1,), (1,)), ((), ())),
        preferred_element_type=jnp.float32)
    acc_ref[...] = acc_ref[...] * alpha[:, 0:1] + pv

    @pl.when(i == n - 1)
    def _finish():
        o_ref[...] = acc_ref[...] / l_ref[...][:, 0:1]


def kernel(x, memory_keys, memory_values):
    kT = memory_keys.T
    vT = memory_values.T
    m_total = memory_keys.shape[0]
    grid = (m_total // _BLOCK,)
    return pl.pallas_call(
        _flash_kernel,
        grid=grid,
        in_specs=[
            pl.BlockSpec((_B, _D), lambda i: (0, 0)),
            pl.BlockSpec((_D, _BLOCK), lambda i: (0, i)),
            pl.BlockSpec((_D, _BLOCK), lambda i: (0, i)),
        ],
        out_specs=pl.BlockSpec((_B, _D), lambda i: (0, 0)),
        out_shape=jax.ShapeDtypeStruct((_B, _D), jnp.float32),
        scratch_shapes=[
            pltpu.VMEM((_B, 128), jnp.float32),
            pltpu.VMEM((_B, 128), jnp.float32),
            pltpu.VMEM((_B, _D), jnp.float32),
        ],
        compiler_params=pltpu.CompilerParams(
            dimension_semantics=("parallel",),
        ),
    )(x, kT, vT)
